# baseline (speedup 1.0000x reference)
import functools

import jax
import jax.numpy as jnp
from jax import lax
from jax.experimental import pallas as pl
from jax.experimental.pallas import tpu as pltpu
from jax.experimental.pallas import tpu_sc as plsc

M = 100000
D = 128
B = 16384

NC = 2
NS = 16
L = 16

CH = 5888
NMAIN = 8
CORE0_BASE = 0
CORE1_BASE = NMAIN * CH + 2944
REM0 = 2944
REM1 = 2816
TAIL = 32
SLICE = B // NS
KB = 128
NB = SLICE // KB
CLEN = SLICE + L
IGNORE = -1

_mesh = plsc.VectorSubcoreMesh(core_axis_name="c", subcore_axis_name="s")


@functools.partial(
    pl.kernel,
    out_type=jax.ShapeDtypeStruct((M, D), jnp.float32),
    mesh=_mesh,
    compiler_params=pltpu.CompilerParams(needs_layout_passes=False),
    scratch_types=(
        pltpu.VMEM((SLICE,), jnp.int32),
        pltpu.VMEM((CLEN,), jnp.int32),
        pltpu.VMEM((CLEN,), jnp.int32),
        pltpu.VMEM((KB, D), jnp.float32),
        [pltpu.VMEM_SHARED((CH, D), jnp.float32) for _ in range(2)],
        pltpu.SemaphoreType.DMA,
        pltpu.SemaphoreType.DMA,
        [pltpu.SemaphoreType.DMA for _ in range(2)],
        [pltpu.SemaphoreType.DMA for _ in range(2)],
    ),
)
def _scatter_add_kernel(x_hbm, idx_hbm, val_hbm, out_hbm,
                        idx_v, pos_c, off_c, row_buf, accs,
                        gsem, ssem, stage_sems, drain_sems):
  cid = lax.axis_index("c")
  sid = lax.axis_index("s")
  iota = lax.iota(jnp.int32, L)
  core_base = jnp.where(cid == 0, CORE0_BASE, CORE1_BASE).astype(jnp.int32)

  slice_base = sid * SLICE
  pltpu.sync_copy(idx_hbm.at[pl.ds(slice_base, SLICE)], idx_v)

  def stage_desc(p, chunk_base, rows):
    per_tile = rows // NS
    start = pl.multiple_of(chunk_base + sid * per_tile, 8)
    return pltpu.make_async_copy(
        x_hbm.at[pl.ds(start, per_tile)],
        accs[p].at[pl.ds(sid * per_tile, per_tile)],
        stage_sems[p],
    )

  def drain_desc(p, chunk_base, rows):
    per_tile = rows // NS
    start = pl.multiple_of(chunk_base + sid * per_tile, 8)
    return pltpu.make_async_copy(
        accs[p].at[pl.ds(sid * per_tile, per_tile)],
        out_hbm.at[pl.ds(start, per_tile)],
        drain_sems[p],
    )

  def filters(chunk_base, rows):
    def _fill(k, _):
      off_c[pl.ds(k * L, L)] = jnp.full((L,), IGNORE, jnp.int32)
      pos_c[pl.ds(k * L, L)] = jnp.full((L,), IGNORE, jnp.int32)
      return 0
    lax.fori_loop(0, CLEN // L, _fill, 0)

    ones = jnp.ones((L,), jnp.int32)
    zeros = jnp.zeros((L,), jnp.int32)

    def _vreg(k, cur):
      o = k * L
      v = idx_v[pl.ds(o, L)]
      hit = (v >= chunk_base) & (v < chunk_base + rows)
      csum = plsc.cumsum(jnp.where(hit, ones, zeros))
      wpos = cur + csum - ones
      plsc.store_scatter(off_c, [wpos], v - chunk_base, mask=hit)
      plsc.store_scatter(pos_c, [wpos], iota + (slice_base + o), mask=hit)
      return cur + plsc.all_reduce_population_count(hit)

    cnt = lax.fori_loop(0, SLICE // L, _vreg, zeros)
    return jnp.max(cnt)

  def waves(p, n):
    def _batch(b, _):
      start = b * KB
      pltpu.async_copy(
          val_hbm.at[plsc.Indices(pos_c.at[pl.ds(start, KB)],
                                  ignored_value=IGNORE)],
          row_buf, gsem).wait()
      pltpu.async_copy(
          row_buf,
          accs[p].at[plsc.Indices(off_c.at[pl.ds(start, KB)],
                                  ignored_value=IGNORE)],
          ssem, add=True).wait()
      return 0
    lax.fori_loop(0, (n + KB - 1) // KB, _batch, 0)

  stage_desc(0, core_base, CH).start()

  def pipeline_step(j, _):
    c0_base = core_base + (2 * j) * CH
    c1_base = c0_base + CH

    n0 = filters(c0_base, CH)
    stage_desc(0, c0_base, CH).wait()
    plsc.subcore_barrier()
    waves(0, n0)
    plsc.subcore_barrier()
    drain_desc(0, c0_base, CH).start()

    @pl.when(j > 0)
    def _():
      drain_desc(1, c0_base - CH, CH).wait()
    stage_desc(1, c1_base, CH).start()

    n1 = filters(c1_base, CH)
    stage_desc(1, c1_base, CH).wait()
    plsc.subcore_barrier()
    waves(1, n1)
    plsc.subcore_barrier()
    drain_desc(1, c1_base, CH).start()

    drain_desc(0, c0_base, CH).wait()
    @pl.when(j < (NMAIN // 2) - 1)
    def _():
      stage_desc(0, c1_base + CH, CH).start()
    return 0

  lax.fori_loop(0, NMAIN // 2, pipeline_step, 0)

  @pl.when(cid == 0)
  def _rem0():
    rbase = CORE0_BASE + NMAIN * CH
    stage_desc(0, rbase, REM0).start()
    n = filters(rbase, REM0)
    stage_desc(0, rbase, REM0).wait()
    plsc.subcore_barrier()
    waves(0, n)
    plsc.subcore_barrier()
    drain_desc(0, rbase, REM0).start()
    drain_desc(0, rbase, REM0).wait()

  @pl.when(cid == 1)
  def _rem1():
    rbase = CORE1_BASE + NMAIN * CH
    stage_desc(0, rbase, REM1).start()
    n = filters(rbase, REM1)
    stage_desc(0, rbase, REM1).wait()
    plsc.subcore_barrier()
    waves(0, n)
    plsc.subcore_barrier()
    drain_desc(0, rbase, REM1).start()
    drain_desc(0, rbase, REM1).wait()

    tbase = rbase + REM1
    plsc.subcore_barrier()

    @pl.when(sid == 0)
    def _():
      pltpu.make_async_copy(x_hbm.at[pl.ds(tbase, TAIL)],
                            accs[0].at[pl.ds(0, TAIL)], stage_sems[0]).start()
    n_tail = filters(tbase, TAIL)

    @pl.when(sid == 0)
    def _():
      pltpu.make_async_copy(x_hbm.at[pl.ds(tbase, TAIL)],
                            accs[0].at[pl.ds(0, TAIL)], stage_sems[0]).wait()
    plsc.subcore_barrier()
    waves(0, n_tail)
    plsc.subcore_barrier()

    @pl.when(sid == 0)
    def _():
      pltpu.sync_copy(accs[0].at[pl.ds(0, TAIL)],
                      out_hbm.at[pl.ds(tbase, TAIL)])

  last_b_base = core_base + (NMAIN - 1) * CH
  drain_desc(1, last_b_base, CH).wait()


def kernel(x, indices, values, accumulate):
  del accumulate
  idx32 = indices.astype(jnp.int32)
  return _scatter_add_kernel(x, idx32, values)


# speedup vs baseline: 1.0140x; 1.0140x over previous
import functools

import jax
import jax.numpy as jnp
from jax import lax
from jax.experimental import pallas as pl
from jax.experimental.pallas import tpu as pltpu
from jax.experimental.pallas import tpu_sc as plsc

M = 100000
D = 128
B = 16384

NC = 2
NS = 16
L = 16

CH = 5888
NMAIN = 8
CORE0_BASE = 0
CORE1_BASE = NMAIN * CH + 2944
REM0 = 2944
REM1 = 2816
TAIL = 32
SLICE = B // NS
KB = 128
NB = SLICE // KB
CLEN = SLICE + L
IGNORE = -1

_mesh = plsc.VectorSubcoreMesh(core_axis_name="c", subcore_axis_name="s")


@functools.partial(
    pl.kernel,
    out_type=jax.ShapeDtypeStruct((M, D), jnp.float32),
    mesh=_mesh,
    compiler_params=pltpu.CompilerParams(needs_layout_passes=False),
    scratch_types=(
        pltpu.VMEM((SLICE,), jnp.int32),
        pltpu.VMEM((CLEN,), jnp.int32),
        pltpu.VMEM((CLEN,), jnp.int32),
        pltpu.VMEM((KB, D), jnp.float32),
        [pltpu.VMEM_SHARED((CH, D), jnp.float32) for _ in range(2)],
        pltpu.SemaphoreType.DMA,
        pltpu.SemaphoreType.DMA,
        [pltpu.SemaphoreType.DMA for _ in range(2)],
        [pltpu.SemaphoreType.DMA for _ in range(2)],
    ),
)
def _scatter_add_kernel(x_hbm, idx_hbm, val_hbm, out_hbm,
                        idx_v, pos_c, off_c, row_buf, accs,
                        gsem, ssem, stage_sems, drain_sems):
  cid = lax.axis_index("c")
  sid = lax.axis_index("s")
  iota = lax.iota(jnp.int32, L)
  core_base = jnp.where(cid == 0, CORE0_BASE, CORE1_BASE).astype(jnp.int32)

  slice_base = sid * SLICE
  pltpu.sync_copy(idx_hbm.at[pl.ds(slice_base, SLICE)], idx_v)

  def stage_desc(p, chunk_base, rows):
    per_tile = rows // NS
    start = pl.multiple_of(chunk_base + sid * per_tile, 8)
    return pltpu.make_async_copy(
        x_hbm.at[pl.ds(start, per_tile)],
        accs[p].at[pl.ds(sid * per_tile, per_tile)],
        stage_sems[p],
    )

  def drain_desc(p, chunk_base, rows):
    per_tile = rows // NS
    start = pl.multiple_of(chunk_base + sid * per_tile, 8)
    return pltpu.make_async_copy(
        accs[p].at[pl.ds(sid * per_tile, per_tile)],
        out_hbm.at[pl.ds(start, per_tile)],
        drain_sems[p],
    )

  def filters(chunk_base, rows):
    def _fill(k, _):
      off_c[pl.ds(k * L, L)] = jnp.full((L,), IGNORE, jnp.int32)
      pos_c[pl.ds(k * L, L)] = jnp.full((L,), IGNORE, jnp.int32)
      return 0
    lax.fori_loop(0, CLEN // L, _fill, 0)

    ones = jnp.ones((L,), jnp.int32)
    zeros = jnp.zeros((L,), jnp.int32)

    def _vreg(k, cur):
      o = k * L
      v = idx_v[pl.ds(o, L)]
      hit = (v >= chunk_base) & (v < chunk_base + rows)
      csum = plsc.cumsum(jnp.where(hit, ones, zeros))
      wpos = cur + csum - ones
      plsc.store_scatter(off_c, [wpos], v - chunk_base, mask=hit)
      plsc.store_scatter(pos_c, [wpos], iota + (slice_base + o), mask=hit)
      return cur + plsc.all_reduce_population_count(hit)

    cnt = lax.fori_loop(0, SLICE // L, _vreg, zeros)
    return jnp.max(cnt)

  def waves(p, n):
    def _batch(b, _):
      start = b * KB
      pltpu.async_copy(
          val_hbm.at[plsc.Indices(pos_c.at[pl.ds(start, KB)],
                                  ignored_value=IGNORE)],
          row_buf, gsem).wait()
      pltpu.async_copy(
          row_buf,
          accs[p].at[plsc.Indices(off_c.at[pl.ds(start, KB)],
                                  ignored_value=IGNORE)],
          ssem, add=True).wait()
      return 0
    lax.fori_loop(0, (n + KB - 1) // KB, _batch, 0)

  stage_desc(0, core_base, CH).start()

  def pipeline_step(j, _):
    c0_base = core_base + (2 * j) * CH
    c1_base = c0_base + CH

    n0 = filters(c0_base, CH)
    stage_desc(0, c0_base, CH).wait()
    plsc.subcore_barrier()

    @pl.when(j > 0)
    def _():
      drain_desc(1, c0_base - CH, CH).wait()
    stage_desc(1, c1_base, CH).start()

    waves(0, n0)
    plsc.subcore_barrier()
    drain_desc(0, c0_base, CH).start()

    n1 = filters(c1_base, CH)
    stage_desc(1, c1_base, CH).wait()
    plsc.subcore_barrier()

    drain_desc(0, c0_base, CH).wait()

    @pl.when(j < (NMAIN // 2) - 1)
    def _():
      stage_desc(0, c1_base + CH, CH).start()

    waves(1, n1)
    plsc.subcore_barrier()
    drain_desc(1, c1_base, CH).start()
    return 0

  lax.fori_loop(0, NMAIN // 2, pipeline_step, 0)

  @pl.when(cid == 0)
  def _rem0():
    rbase = CORE0_BASE + NMAIN * CH
    stage_desc(0, rbase, REM0).start()
    n = filters(rbase, REM0)
    stage_desc(0, rbase, REM0).wait()
    plsc.subcore_barrier()
    waves(0, n)
    plsc.subcore_barrier()
    drain_desc(0, rbase, REM0).start()
    drain_desc(0, rbase, REM0).wait()

  @pl.when(cid == 1)
  def _rem1():
    rbase = CORE1_BASE + NMAIN * CH
    stage_desc(0, rbase, REM1).start()
    n = filters(rbase, REM1)
    stage_desc(0, rbase, REM1).wait()
    plsc.subcore_barrier()
    waves(0, n)
    plsc.subcore_barrier()
    drain_desc(0, rbase, REM1).start()
    drain_desc(0, rbase, REM1).wait()

    tbase = rbase + REM1
    plsc.subcore_barrier()

    @pl.when(sid == 0)
    def _():
      pltpu.make_async_copy(x_hbm.at[pl.ds(tbase, TAIL)],
                            accs[0].at[pl.ds(0, TAIL)], stage_sems[0]).start()
    n_tail = filters(tbase, TAIL)

    @pl.when(sid == 0)
    def _():
      pltpu.make_async_copy(x_hbm.at[pl.ds(tbase, TAIL)],
                            accs[0].at[pl.ds(0, TAIL)], stage_sems[0]).wait()
    plsc.subcore_barrier()
    waves(0, n_tail)
    plsc.subcore_barrier()

    @pl.when(sid == 0)
    def _():
      pltpu.sync_copy(accs[0].at[pl.ds(0, TAIL)],
                      out_hbm.at[pl.ds(tbase, TAIL)])

  last_b_base = core_base + (NMAIN - 1) * CH
  drain_desc(1, last_b_base, CH).wait()


def kernel(x, indices, values, accumulate):
  del accumulate
  idx32 = indices.astype(jnp.int32)
  return _scatter_add_kernel(x, idx32, values)
